# baseline (speedup 1.0000x reference)
import functools

import jax
import jax.numpy as jnp
import numpy as np
from jax import lax
from jax.experimental import pallas as pl
from jax.experimental.pallas import tpu as pltpu
from jax.experimental.pallas import tpu_sc as plsc

BS = 16384
DIM = 128
DSMALL = 10
NROLES = 1639
NPROP = 18
NARG = 3
NQ = 6
T3ROWS = 6 * 64

NC, NS = 2, 16
NW = NC * NS
BPW = BS // NW
RCH = 64
OCB = 64


def _build_csel():
    c = np.zeros((T3ROWS, 4 * NPROP), np.float32)
    for q in range(NQ):
        for comb in range(64):
            a, b, g = comb & 3, (comb >> 2) & 3, (comb >> 4) & 3
            c[64 * q + comb, 4 * (3 * q) + a] += 1.0
            c[64 * q + comb, 4 * (3 * q + 1) + b] += 1.0
            c[64 * q + comb, 4 * (3 * q + 2) + g] += 1.0
    return c


def _build_cidx_mat():
    m = np.zeros((NARG * NPROP, 32), np.float32)
    off = np.zeros((1, 32), np.float32)
    for a in range(NARG):
        for q in range(NQ):
            for j in range(3):
                m[18 * a + 3 * q + j, 8 * a + q] = float(4 ** j)
            off[0, 8 * a + q] = float(64 * q)
    return m, off


_CSEL = _build_csel()
_CIDX_M, _CIDX_OFF = _build_cidx_mat()


def _prep_body(role_ref, wr_ref, br_ref, posr_ref, obj_ref, wt_ref, bt_ref,
               csel_ref, lnb_ref, poso_ref, rtab_ref, t3_ref, b8_ref, a2_ref):
    rt = role_ref[...]
    wr = wr_ref[...]
    rtab = lax.dot_general(rt, wr, (((1,), (1,)), ((), ())),
                           preferred_element_type=jnp.float32)
    rtab_ref[...] = rtab + br_ref[...] + posr_ref[...]

    obj = obj_ref[...]
    a2_ref[...] = jnp.zeros((4 * NPROP, NPROP * DIM), jnp.float32)
    for p in range(NPROP):
        a2_ref[4 * p:4 * p + 4, DIM * p:DIM * (p + 1)] = obj[p:p + 4, :]
    t2 = lax.dot_general(a2_ref[...], wt_ref[...], (((1,), (1,)), ((), ())),
                         preferred_element_type=jnp.float32)
    t3 = lax.dot_general(csel_ref[...], t2, (((1,), (0,)), ((), ())),
                         preferred_element_type=jnp.float32)
    t3_ref[...] = t3 + bt_ref[...] * (1.0 / 6.0)

    b3 = poso_ref[...] + lnb_ref[...]
    b8_ref[...] = jnp.concatenate([b3, jnp.zeros((5, DIM), jnp.float32)], axis=0)


LNBLK = 1024


def _ln_body(s_ref, g_ref, b8_ref, out_ref):
    x = s_ref[...]
    x = jnp.maximum(x, 0.0)
    mu = jnp.mean(x, axis=-1, keepdims=True)
    xc = x - mu
    var = jnp.mean(xc * xc, axis=-1, keepdims=True)
    y = xc * jax.lax.rsqrt(var + 1e-5) * g_ref[...]
    y3 = y.reshape(LNBLK, NARG, DIM) + b8_ref[0:NARG, :][None, :, :]
    out_ref[...] = y3.reshape(NARG * LNBLK, DIM)


def _cidx_body(p54_ref, m_ref, off_ref, out_ref):
    x = p54_ref[...].astype(jnp.float32)
    c = lax.dot_general(x, m_ref[...], (((1,), (0,)), ((), ())),
                        preferred_element_type=jnp.float32)
    out_ref[...] = (c + off_ref[...]).astype(jnp.int32)


def _sc_body(rtab, roleset, t3f, cidx, role_out, obj_out,
             t3_v, ridx_b, rrows_b, cidx_b, obj_b,
             sem_i, sem_g, sem_w, sem_c, sem_o):
    wid = lax.axis_index("s") * NC + lax.axis_index("c")
    b0 = wid * BPW
    nch = BPW // OCB

    pltpu.sync_copy(t3f, t3_v)

    iota = lax.iota(jnp.int32, 16)
    cols = [iota + 16 * j for j in range(8)]

    def rbase(c):
        return b0 + c * RCH

    idx_d = {
        0: pltpu.async_copy(roleset.at[pl.ds(rbase(0), RCH)], ridx_b[0], sem_i[0]),
        1: pltpu.async_copy(roleset.at[pl.ds(rbase(1), RCH)], ridx_b[1], sem_i[1]),
    }
    cidx_d = {0: pltpu.async_copy(cidx.at[pl.ds(32 * b0, 32 * OCB)],
                                  cidx_b[0], sem_c[0])}
    gather_d, wb_d, obj_d = {}, {}, {}

    for i in range(nch):
        bb = b0 + i * OCB
        p = i % 2
        if i >= 1:
            gather_d[i - 1].wait()
            wb_d[i - 1] = pltpu.async_copy(
                rrows_b[(i - 1) % 2], role_out.at[pl.ds(rbase(i - 1), RCH)],
                sem_w[(i - 1) % 2])
            if i + 1 < nch:
                idx_d[i + 1] = pltpu.async_copy(
                    roleset.at[pl.ds(rbase(i + 1), RCH)],
                    ridx_b[(i + 1) % 2], sem_i[(i + 1) % 2])
        if i >= 2:
            wb_d[i - 2].wait()
        idx_d[i].wait()
        gather_d[i] = pltpu.async_copy(rtab.at[ridx_b[p]], rrows_b[p], sem_g[p])
        if i + 1 < nch:
            cidx_d[i + 1] = pltpu.async_copy(
                cidx.at[pl.ds(32 * (bb + OCB), 32 * OCB)],
                cidx_b[(i + 1) % 2], sem_c[(i + 1) % 2])
        if i >= 2:
            obj_d[i - 2].wait()
        cidx_d[i].wait()
        cidx_v = cidx_b[p]
        obj_v = obj_b[p]

        @plsc.parallel_loop(0, OCB, 1, unroll=2)
        def one_b(lb):
            base32 = 32 * lb
            for a in range(NARG):
                accs = [None] * 8
                for q in range(NQ):
                    addr = jnp.broadcast_to(base32 + (8 * a + q), (16,))
                    row = plsc.load_gather(cidx_v, [addr])
                    for j in range(8):
                        g = plsc.load_gather(t3_v, [row, cols[j]])
                        accs[j] = g if q == 0 else accs[j] + g
                orow = 3 * lb + a
                for j in range(8):
                    obj_v[orow, pl.ds(16 * j, 16)] = accs[j]

        obj_d[i] = pltpu.async_copy(
            obj_v, obj_out.at[pl.ds(NARG * bb, NARG * OCB)], sem_o[p])

    gather_d[nch - 1].wait()
    wb_d[nch - 1] = pltpu.async_copy(
        rrows_b[(nch - 1) % 2], role_out.at[pl.ds(rbase(nch - 1), RCH)],
        sem_w[(nch - 1) % 2])
    wb_d[nch - 2].wait()
    wb_d[nch - 1].wait()
    obj_d[nch - 2].wait()
    obj_d[nch - 1].wait()


def kernel(roleset, properties, role_table, Wr, br, obj_table, Wt, bt,
           ln_gamma, ln_beta, pos_role, pos_obj):
    roleset = roleset.astype(jnp.int32)
    properties = properties.astype(jnp.int32)

    rtab, t3f, beta8 = pl.pallas_call(
        _prep_body,
        out_shape=(
            jax.ShapeDtypeStruct((NROLES + 1, DIM), jnp.float32),
            jax.ShapeDtypeStruct((T3ROWS, DIM), jnp.float32),
            jax.ShapeDtypeStruct((8, DIM), jnp.float32),
        ),
        scratch_shapes=[pltpu.VMEM((4 * NPROP, NPROP * DIM), jnp.float32)],
    )(role_table, Wr, br.reshape(1, DIM), pos_role, obj_table, Wt,
      bt.reshape(1, DIM), jnp.asarray(_CSEL), ln_beta.reshape(1, DIM),
      pos_obj.reshape(NARG, DIM))

    p54 = properties.reshape(BS, NARG * NPROP)
    nblk = 16
    blk = BS // nblk
    cidx = pl.pallas_call(
        _cidx_body,
        grid=(nblk,),
        in_specs=[
            pl.BlockSpec((blk, NARG * NPROP), lambda i: (i, 0)),
            pl.BlockSpec((NARG * NPROP, 32), lambda i: (0, 0)),
            pl.BlockSpec((1, 32), lambda i: (0, 0)),
        ],
        out_specs=pl.BlockSpec((blk, 32), lambda i: (i, 0)),
        out_shape=jax.ShapeDtypeStruct((BS, 32), jnp.int32),
    )(p54, jnp.asarray(_CIDX_M), jnp.asarray(_CIDX_OFF))

    mesh = plsc.VectorSubcoreMesh(core_axis_name="c", subcore_axis_name="s",
                                  num_cores=NC, num_subcores=NS)
    sc = pl.kernel(
        _sc_body,
        out_type=(
            jax.ShapeDtypeStruct((BS, DIM), jnp.float32),
            jax.ShapeDtypeStruct((BS * NARG, DIM), jnp.float32),
        ),
        mesh=mesh,
        compiler_params=pltpu.CompilerParams(needs_layout_passes=False),
        scratch_types=[
            pltpu.VMEM((T3ROWS, DIM), jnp.float32),
            [pltpu.VMEM((RCH,), jnp.int32) for _ in range(2)],
            [pltpu.VMEM((RCH, DIM), jnp.float32) for _ in range(2)],
            [pltpu.VMEM((OCB * 32,), jnp.int32) for _ in range(2)],
            [pltpu.VMEM((NARG * OCB, DIM), jnp.float32) for _ in range(2)],
            [pltpu.SemaphoreType.DMA for _ in range(2)],
            [pltpu.SemaphoreType.DMA for _ in range(2)],
            [pltpu.SemaphoreType.DMA for _ in range(2)],
            [pltpu.SemaphoreType.DMA for _ in range(2)],
            [pltpu.SemaphoreType.DMA for _ in range(2)],
        ],
    )
    role_out, s_flat = sc(rtab, roleset, t3f, cidx.reshape(-1))

    obj = pl.pallas_call(
        _ln_body,
        grid=(BS // LNBLK,),
        in_specs=[
            pl.BlockSpec((NARG * LNBLK, DIM), lambda i: (i, 0)),
            pl.BlockSpec((1, DIM), lambda i: (0, 0)),
            pl.BlockSpec((8, DIM), lambda i: (0, 0)),
        ],
        out_specs=pl.BlockSpec((NARG * LNBLK, DIM), lambda i: (i, 0)),
        out_shape=jax.ShapeDtypeStruct((NARG * BS, DIM), jnp.float32),
    )(s_flat, ln_gamma.reshape(1, DIM), beta8)
    return role_out, obj.reshape(BS, NARG, DIM)


# speedup vs baseline: 1.2023x; 1.2023x over previous
import functools

import jax
import jax.numpy as jnp
import numpy as np
from jax import lax
from jax.experimental import pallas as pl
from jax.experimental.pallas import tpu as pltpu
from jax.experimental.pallas import tpu_sc as plsc

BS = 16384
DIM = 128
DSMALL = 10
NROLES = 1639
NPROP = 18
NARG = 3
NQ = 6
T3ROWS = 6 * 64

NC, NS = 2, 16
NW = NC * NS
BPW = BS // NW
RCH = 64
OCB = 64


def _build_csel():
    c = np.zeros((T3ROWS, 4 * NPROP), np.float32)
    for q in range(NQ):
        for comb in range(64):
            a, b, g = comb & 3, (comb >> 2) & 3, (comb >> 4) & 3
            c[64 * q + comb, 4 * (3 * q) + a] += 1.0
            c[64 * q + comb, 4 * (3 * q + 1) + b] += 1.0
            c[64 * q + comb, 4 * (3 * q + 2) + g] += 1.0
    return c


def _build_cidx_mat():
    m = np.zeros((NARG * NPROP, 32), np.float32)
    off = np.zeros((1, 32), np.float32)
    for a in range(NARG):
        for q in range(NQ):
            for j in range(3):
                m[18 * a + 3 * q + j, 8 * a + q] = float(4 ** j)
            off[0, 8 * a + q] = float(64 * q)
    return m, off


_CSEL = _build_csel()
_CIDX_M, _CIDX_OFF = _build_cidx_mat()


def _prep_body(role_ref, wr_ref, br_ref, posr_ref, obj_ref, wt_ref, bt_ref,
               csel_ref, lnb_ref, poso_ref, rtab_ref, t3_ref, b8_ref, a2_ref):
    rt = role_ref[...]
    wr = wr_ref[...]
    rtab = lax.dot_general(rt, wr, (((1,), (1,)), ((), ())),
                           preferred_element_type=jnp.float32)
    rtab_ref[...] = rtab + br_ref[...] + posr_ref[...]

    obj = obj_ref[...]
    a2_ref[...] = jnp.zeros((4 * NPROP, NPROP * DIM), jnp.float32)
    for p in range(NPROP):
        a2_ref[4 * p:4 * p + 4, DIM * p:DIM * (p + 1)] = obj[p:p + 4, :]
    t2 = lax.dot_general(a2_ref[...], wt_ref[...], (((1,), (1,)), ((), ())),
                         preferred_element_type=jnp.float32)
    t3 = lax.dot_general(csel_ref[...], t2, (((1,), (0,)), ((), ())),
                         preferred_element_type=jnp.float32)
    t3_ref[...] = t3 + bt_ref[...] * (1.0 / 6.0)

    b3 = poso_ref[...] + lnb_ref[...]
    b8_ref[...] = jnp.concatenate([b3, jnp.zeros((5, DIM), jnp.float32)], axis=0)


LNBLK = 1024


def _ln_body(s_ref, g_ref, b8_ref, out_ref):
    x = s_ref[...]
    x = jnp.maximum(x, 0.0)
    mu = jnp.mean(x, axis=-1, keepdims=True)
    xc = x - mu
    var = jnp.mean(xc * xc, axis=-1, keepdims=True)
    y = xc * jax.lax.rsqrt(var + 1e-5) * g_ref[...]
    y3 = y.reshape(LNBLK, NARG, DIM) + b8_ref[0:NARG, :][None, :, :]
    out_ref[...] = y3


def _cidx_body(p54_ref, m_ref, off_ref, out_ref):
    x = p54_ref[...].astype(jnp.float32)
    c = lax.dot_general(x, m_ref[...], (((1,), (0,)), ((), ())),
                        preferred_element_type=jnp.float32)
    out_ref[...] = (c + off_ref[...]).astype(jnp.int32)


def _sc_body(rtab, roleset, t3f, cidx, role_out, obj_out,
             t3_v, ridx_b, rrows_b, cidx_b, obj_b,
             sem_i, sem_g, sem_w, sem_c, sem_o):
    wid = lax.axis_index("s") * NC + lax.axis_index("c")
    b0 = wid * BPW
    nch = BPW // OCB

    pltpu.sync_copy(t3f, t3_v)

    iota = lax.iota(jnp.int32, 16)
    cols = [iota + 16 * j for j in range(8)]

    def rbase(c):
        return b0 + c * RCH

    idx_d = {
        0: pltpu.async_copy(roleset.at[pl.ds(rbase(0), RCH)], ridx_b[0], sem_i[0]),
        1: pltpu.async_copy(roleset.at[pl.ds(rbase(1), RCH)], ridx_b[1], sem_i[1]),
    }
    cidx_d = {0: pltpu.async_copy(cidx.at[pl.ds(32 * b0, 32 * OCB)],
                                  cidx_b[0], sem_c[0])}
    gather_d, wb_d, obj_d = {}, {}, {}

    for i in range(nch):
        bb = b0 + i * OCB
        p = i % 2
        if i >= 1:
            gather_d[i - 1].wait()
            wb_d[i - 1] = pltpu.async_copy(
                rrows_b[(i - 1) % 2], role_out.at[pl.ds(rbase(i - 1), RCH)],
                sem_w[(i - 1) % 2])
            if i + 1 < nch:
                idx_d[i + 1] = pltpu.async_copy(
                    roleset.at[pl.ds(rbase(i + 1), RCH)],
                    ridx_b[(i + 1) % 2], sem_i[(i + 1) % 2])
        if i >= 2:
            wb_d[i - 2].wait()
        idx_d[i].wait()
        gather_d[i] = pltpu.async_copy(rtab.at[ridx_b[p]], rrows_b[p], sem_g[p])
        if i + 1 < nch:
            cidx_d[i + 1] = pltpu.async_copy(
                cidx.at[pl.ds(32 * (bb + OCB), 32 * OCB)],
                cidx_b[(i + 1) % 2], sem_c[(i + 1) % 2])
        if i >= 2:
            obj_d[i - 2].wait()
        cidx_d[i].wait()
        cidx_v = cidx_b[p]
        obj_v = obj_b[p]

        @plsc.parallel_loop(0, OCB, 1, unroll=2)
        def one_b(lb):
            base32 = 32 * lb
            for a in range(NARG):
                accs = [None] * 8
                for q in range(NQ):
                    addr = jnp.broadcast_to(base32 + (8 * a + q), (16,))
                    row = plsc.load_gather(cidx_v, [addr])
                    for j in range(8):
                        g = plsc.load_gather(t3_v, [row, cols[j]])
                        accs[j] = g if q == 0 else accs[j] + g
                orow = 3 * lb + a
                for j in range(8):
                    obj_v[orow, pl.ds(16 * j, 16)] = accs[j]

        obj_d[i] = pltpu.async_copy(
            obj_v, obj_out.at[pl.ds(NARG * bb, NARG * OCB)], sem_o[p])

    gather_d[nch - 1].wait()
    wb_d[nch - 1] = pltpu.async_copy(
        rrows_b[(nch - 1) % 2], role_out.at[pl.ds(rbase(nch - 1), RCH)],
        sem_w[(nch - 1) % 2])
    wb_d[nch - 2].wait()
    wb_d[nch - 1].wait()
    obj_d[nch - 2].wait()
    obj_d[nch - 1].wait()


def kernel(roleset, properties, role_table, Wr, br, obj_table, Wt, bt,
           ln_gamma, ln_beta, pos_role, pos_obj):
    roleset = roleset.astype(jnp.int32)
    properties = properties.astype(jnp.int32)

    rtab, t3f, beta8 = pl.pallas_call(
        _prep_body,
        out_shape=(
            jax.ShapeDtypeStruct((NROLES + 1, DIM), jnp.float32),
            jax.ShapeDtypeStruct((T3ROWS, DIM), jnp.float32),
            jax.ShapeDtypeStruct((8, DIM), jnp.float32),
        ),
        scratch_shapes=[pltpu.VMEM((4 * NPROP, NPROP * DIM), jnp.float32)],
    )(role_table, Wr, br.reshape(1, DIM), pos_role, obj_table, Wt,
      bt.reshape(1, DIM), jnp.asarray(_CSEL), ln_beta.reshape(1, DIM),
      pos_obj.reshape(NARG, DIM))

    p54 = properties.reshape(BS, NARG * NPROP)
    nblk = 16
    blk = BS // nblk
    cidx = pl.pallas_call(
        _cidx_body,
        grid=(nblk,),
        in_specs=[
            pl.BlockSpec((blk, NARG * NPROP), lambda i: (i, 0)),
            pl.BlockSpec((NARG * NPROP, 32), lambda i: (0, 0)),
            pl.BlockSpec((1, 32), lambda i: (0, 0)),
        ],
        out_specs=pl.BlockSpec((blk, 32), lambda i: (i, 0)),
        out_shape=jax.ShapeDtypeStruct((BS, 32), jnp.int32),
    )(p54, jnp.asarray(_CIDX_M), jnp.asarray(_CIDX_OFF))

    mesh = plsc.VectorSubcoreMesh(core_axis_name="c", subcore_axis_name="s",
                                  num_cores=NC, num_subcores=NS)
    sc = pl.kernel(
        _sc_body,
        out_type=(
            jax.ShapeDtypeStruct((BS, DIM), jnp.float32),
            jax.ShapeDtypeStruct((BS * NARG, DIM), jnp.float32),
        ),
        mesh=mesh,
        compiler_params=pltpu.CompilerParams(needs_layout_passes=False),
        scratch_types=[
            pltpu.VMEM((T3ROWS, DIM), jnp.float32),
            [pltpu.VMEM((RCH,), jnp.int32) for _ in range(2)],
            [pltpu.VMEM((RCH, DIM), jnp.float32) for _ in range(2)],
            [pltpu.VMEM((OCB * 32,), jnp.int32) for _ in range(2)],
            [pltpu.VMEM((NARG * OCB, DIM), jnp.float32) for _ in range(2)],
            [pltpu.SemaphoreType.DMA for _ in range(2)],
            [pltpu.SemaphoreType.DMA for _ in range(2)],
            [pltpu.SemaphoreType.DMA for _ in range(2)],
            [pltpu.SemaphoreType.DMA for _ in range(2)],
            [pltpu.SemaphoreType.DMA for _ in range(2)],
        ],
    )
    role_out, s_flat = sc(rtab, roleset, t3f, cidx.reshape(-1))

    obj = pl.pallas_call(
        _ln_body,
        grid=(BS // LNBLK,),
        in_specs=[
            pl.BlockSpec((NARG * LNBLK, DIM), lambda i: (i, 0)),
            pl.BlockSpec((1, DIM), lambda i: (0, 0)),
            pl.BlockSpec((8, DIM), lambda i: (0, 0)),
        ],
        out_specs=pl.BlockSpec((LNBLK, NARG, DIM), lambda i: (i, 0, 0)),
        out_shape=jax.ShapeDtypeStruct((BS, NARG, DIM), jnp.float32),
    )(s_flat, ln_gamma.reshape(1, DIM), beta8)
    return role_out, obj


# speedup vs baseline: 1.2396x; 1.0310x over previous
import functools

import jax
import jax.numpy as jnp
import numpy as np
from jax import lax
from jax.experimental import pallas as pl
from jax.experimental.pallas import tpu as pltpu
from jax.experimental.pallas import tpu_sc as plsc

BS = 16384
DIM = 128
DSMALL = 10
NROLES = 1639
NPROP = 18
NARG = 3
NQ = 6
T3ROWS = 6 * 64

NC, NS = 2, 16
NW = NC * NS
BPW = BS // NW
RCH = 64
OCB = 64


def _build_csel():
    c = np.zeros((T3ROWS, 4 * NPROP), np.float32)
    for q in range(NQ):
        for comb in range(64):
            a, b, g = comb & 3, (comb >> 2) & 3, (comb >> 4) & 3
            c[64 * q + comb, 4 * (3 * q) + a] += 1.0
            c[64 * q + comb, 4 * (3 * q + 1) + b] += 1.0
            c[64 * q + comb, 4 * (3 * q + 2) + g] += 1.0
    return c


def _build_cidx_mat():
    m = np.zeros((NARG * NPROP, 32), np.float32)
    off = np.zeros((1, 32), np.float32)
    for a in range(NARG):
        for q in range(NQ):
            for j in range(3):
                m[18 * a + 3 * q + j, 8 * a + q] = float(4 ** j)
            off[0, 8 * a + q] = float(64 * q)
    return m, off


_CSEL = _build_csel()
_CIDX_M, _CIDX_OFF = _build_cidx_mat()


def _prep_body(role_ref, wr_ref, br_ref, posr_ref, obj_ref, wt_ref, bt_ref,
               csel_ref, lnb_ref, poso_ref, rtab_ref, t3_ref, b8_ref, a2_ref):
    rt = role_ref[...]
    wr = wr_ref[...]
    rtab = lax.dot_general(rt, wr, (((1,), (1,)), ((), ())),
                           preferred_element_type=jnp.float32)
    rtab_ref[...] = rtab + br_ref[...] + posr_ref[...]

    obj = obj_ref[...]
    a2_ref[...] = jnp.zeros((4 * NPROP, NPROP * DIM), jnp.float32)
    for p in range(NPROP):
        a2_ref[4 * p:4 * p + 4, DIM * p:DIM * (p + 1)] = obj[p:p + 4, :]
    t2 = lax.dot_general(a2_ref[...], wt_ref[...], (((1,), (1,)), ((), ())),
                         preferred_element_type=jnp.float32)
    t3 = lax.dot_general(csel_ref[...], t2, (((1,), (0,)), ((), ())),
                         preferred_element_type=jnp.float32)
    t3_ref[...] = t3 + bt_ref[...] * (1.0 / 6.0)

    b3 = poso_ref[...] + lnb_ref[...]
    b8_ref[...] = jnp.concatenate([b3, jnp.zeros((5, DIM), jnp.float32)], axis=0)


LNBLK = 1024


def _ln_body(s_ref, g_ref, b8_ref, out_ref):
    x = s_ref[...]
    x = jnp.maximum(x, 0.0)
    mu = jnp.mean(x, axis=-1, keepdims=True)
    xc = x - mu
    var = jnp.mean(xc * xc, axis=-1, keepdims=True)
    y = xc * jax.lax.rsqrt(var + 1e-5) * g_ref[...]
    y3 = y.reshape(LNBLK, NARG, DIM) + b8_ref[0:NARG, :][None, :, :]
    out_ref[...] = y3


def _cidx_body(p54_ref, m_ref, off_ref, out_ref):
    x = p54_ref[...].astype(jnp.float32)
    c = lax.dot_general(x, m_ref[...], (((1,), (0,)), ((), ())),
                        preferred_element_type=jnp.float32)
    out_ref[...] = (c + off_ref[...]).astype(jnp.int32)


def _sc_body(rtab, roleset, t3f, cidx, role_out, obj_out,
             t3_v, ridx_b, rrows_b, cidx_b, obj_b,
             sem_i, sem_g, sem_w, sem_c, sem_o):
    wid = lax.axis_index("s") * NC + lax.axis_index("c")
    b0 = wid * BPW
    nch = BPW // OCB

    pltpu.sync_copy(t3f, t3_v)

    iota = lax.iota(jnp.int32, 16)
    cols = [iota + 16 * j for j in range(8)]

    def rbase(c):
        return b0 + c * RCH

    idx_d = {
        0: pltpu.async_copy(roleset.at[pl.ds(rbase(0), RCH)], ridx_b[0], sem_i[0]),
        1: pltpu.async_copy(roleset.at[pl.ds(rbase(1), RCH)], ridx_b[1], sem_i[1]),
    }
    cidx_d = {0: pltpu.async_copy(cidx.at[pl.ds(32 * b0, 32 * OCB)],
                                  cidx_b[0], sem_c[0])}
    gather_d, wb_d, obj_d = {}, {}, {}

    for i in range(nch):
        bb = b0 + i * OCB
        p = i % 2
        if i >= 1:
            gather_d[i - 1].wait()
            wb_d[i - 1] = pltpu.async_copy(
                rrows_b[(i - 1) % 2], role_out.at[pl.ds(rbase(i - 1), RCH)],
                sem_w[(i - 1) % 2])
            if i + 1 < nch:
                idx_d[i + 1] = pltpu.async_copy(
                    roleset.at[pl.ds(rbase(i + 1), RCH)],
                    ridx_b[(i + 1) % 2], sem_i[(i + 1) % 2])
        if i >= 2:
            wb_d[i - 2].wait()
        idx_d[i].wait()
        gather_d[i] = pltpu.async_copy(rtab.at[ridx_b[p]], rrows_b[p], sem_g[p])
        if i + 1 < nch:
            cidx_d[i + 1] = pltpu.async_copy(
                cidx.at[pl.ds(32 * (bb + OCB), 32 * OCB)],
                cidx_b[(i + 1) % 2], sem_c[(i + 1) % 2])
        if i >= 2:
            obj_d[i - 2].wait()
        cidx_d[i].wait()
        cidx_v = cidx_b[p]
        obj_v = obj_b[p]

        @plsc.parallel_loop(0, OCB, 1, unroll=2)
        def one_b(lb):
            base32 = 32 * lb
            for a in range(NARG):
                accs = [None] * 8
                for q in range(NQ):
                    addr = jnp.broadcast_to(base32 + (8 * a + q), (16,))
                    row = plsc.load_gather(cidx_v, [addr])
                    for j in range(8):
                        g = plsc.load_gather(t3_v, [row, cols[j]])
                        accs[j] = g if q == 0 else accs[j] + g
                orow = 3 * lb + a
                for j in range(8):
                    obj_v[orow, pl.ds(16 * j, 16)] = accs[j]

        obj_d[i] = pltpu.async_copy(
            obj_v, obj_out.at[pl.ds(NARG * bb, NARG * OCB)], sem_o[p])

    gather_d[nch - 1].wait()
    wb_d[nch - 1] = pltpu.async_copy(
        rrows_b[(nch - 1) % 2], role_out.at[pl.ds(rbase(nch - 1), RCH)],
        sem_w[(nch - 1) % 2])
    wb_d[nch - 2].wait()
    wb_d[nch - 1].wait()
    obj_d[nch - 2].wait()
    obj_d[nch - 1].wait()


def kernel(roleset, properties, role_table, Wr, br, obj_table, Wt, bt,
           ln_gamma, ln_beta, pos_role, pos_obj):
    roleset = roleset.astype(jnp.int32)
    properties = properties.astype(jnp.int32)

    rtab, t3f, beta8 = pl.pallas_call(
        _prep_body,
        out_shape=(
            jax.ShapeDtypeStruct((NROLES + 1, DIM), jnp.float32),
            jax.ShapeDtypeStruct((T3ROWS, DIM), jnp.float32),
            jax.ShapeDtypeStruct((8, DIM), jnp.float32),
        ),
        scratch_shapes=[pltpu.VMEM((4 * NPROP, NPROP * DIM), jnp.float32)],
    )(role_table, Wr, br.reshape(1, DIM), pos_role, obj_table, Wt,
      bt.reshape(1, DIM), jnp.asarray(_CSEL), ln_beta.reshape(1, DIM),
      pos_obj.reshape(NARG, DIM))

    p54 = properties.reshape(BS, NARG * NPROP)
    nblk = 4
    blk = BS // nblk
    cidx = pl.pallas_call(
        _cidx_body,
        grid=(nblk,),
        in_specs=[
            pl.BlockSpec((blk, NARG * NPROP), lambda i: (i, 0)),
            pl.BlockSpec((NARG * NPROP, 32), lambda i: (0, 0)),
            pl.BlockSpec((1, 32), lambda i: (0, 0)),
        ],
        out_specs=pl.BlockSpec((blk, 32), lambda i: (i, 0)),
        out_shape=jax.ShapeDtypeStruct((BS, 32), jnp.int32),
    )(p54, jnp.asarray(_CIDX_M), jnp.asarray(_CIDX_OFF))

    mesh = plsc.VectorSubcoreMesh(core_axis_name="c", subcore_axis_name="s",
                                  num_cores=NC, num_subcores=NS)
    sc = pl.kernel(
        _sc_body,
        out_type=(
            jax.ShapeDtypeStruct((BS, DIM), jnp.float32),
            jax.ShapeDtypeStruct((BS * NARG, DIM), jnp.float32),
        ),
        mesh=mesh,
        compiler_params=pltpu.CompilerParams(needs_layout_passes=False),
        scratch_types=[
            pltpu.VMEM((T3ROWS, DIM), jnp.float32),
            [pltpu.VMEM((RCH,), jnp.int32) for _ in range(2)],
            [pltpu.VMEM((RCH, DIM), jnp.float32) for _ in range(2)],
            [pltpu.VMEM((OCB * 32,), jnp.int32) for _ in range(2)],
            [pltpu.VMEM((NARG * OCB, DIM), jnp.float32) for _ in range(2)],
            [pltpu.SemaphoreType.DMA for _ in range(2)],
            [pltpu.SemaphoreType.DMA for _ in range(2)],
            [pltpu.SemaphoreType.DMA for _ in range(2)],
            [pltpu.SemaphoreType.DMA for _ in range(2)],
            [pltpu.SemaphoreType.DMA for _ in range(2)],
        ],
    )
    role_out, s_flat = sc(rtab, roleset, t3f, cidx.reshape(-1))

    obj = pl.pallas_call(
        _ln_body,
        grid=(BS // LNBLK,),
        in_specs=[
            pl.BlockSpec((NARG * LNBLK, DIM), lambda i: (i, 0)),
            pl.BlockSpec((1, DIM), lambda i: (0, 0)),
            pl.BlockSpec((8, DIM), lambda i: (0, 0)),
        ],
        out_specs=pl.BlockSpec((LNBLK, NARG, DIM), lambda i: (i, 0, 0)),
        out_shape=jax.ShapeDtypeStruct((BS, NARG, DIM), jnp.float32),
    )(s_flat, ln_gamma.reshape(1, DIM), beta8)
    return role_out, obj
